# Initial kernel scaffold; baseline (speedup 1.0000x reference)
import jax
import jax.numpy as jnp
from jax.experimental import pallas as pl


def kernel(x, edge_index, W1, b1, W2, b2, Wd, bd):
    raise NotImplementedError("write your pallas kernel here")


# speedup vs baseline: 12.6741x; 12.6741x over previous
import functools

import jax
import jax.numpy as jnp
from jax import lax
from jax.experimental import pallas as pl
from jax.experimental.pallas import tpu as pltpu
from jax.experimental.pallas import tpu_sc as plsc

N = 10000
E = 320000
D = 128
NC = 2
NS = 16
CH = 128
NCH = E // CH
NWORK = NC * NS
ITERS = (NCH + NWORK - 1) // NWORK
RPT = 624
NTAIL = N - NS * RPT

_mesh = plsc.VectorSubcoreMesh(core_axis_name="c", subcore_axis_name="s")


def _deg_body(dst_hbm, out_hbm, dst_v, hist):
    c = lax.axis_index("c")
    s = lax.axis_index("s")
    wid = c * NS + s

    def fill_z(i, carry):
        hist[pl.ds(i * 16, 16)] = jnp.zeros((16,), jnp.float32)
        return carry

    lax.fori_loop(0, N // 16, fill_z, 0)

    ones16 = jnp.ones((16,), jnp.float32)

    def step(j, carry):
        chunk = wid + j * NWORK

        @pl.when(chunk < NCH)
        def _():
            pltpu.sync_copy(dst_hbm.at[pl.ds(chunk * CH, CH)], dst_v)
            for k in range(CH // 16):
                idx = dst_v[pl.ds(k * 16, 16)]
                plsc.addupdate_scatter(hist, [idx], ones16)

        return carry

    lax.fori_loop(0, ITERS, step, 0)
    pltpu.sync_copy(hist, out_hbm.at[pl.ds(wid * N, N)])


_deg_kernel = pl.kernel(
    _deg_body,
    out_type=jax.ShapeDtypeStruct((NWORK * N,), jnp.float32),
    mesh=_mesh,
    scratch_types=[
        pltpu.VMEM((CH,), jnp.int32),
        pltpu.VMEM((N,), jnp.float32),
    ],
    compiler_params=pltpu.CompilerParams(needs_layout_passes=False),
)


def _scat_body(src_hbm, dst_hbm, y_hbm, out_hbm, src_v, dst_v, rows_v, zb,
               acc, sem):
    c = lax.axis_index("c")
    s = lax.axis_index("s")
    wid = c * NS + s

    def fill_z(t, carry):
        zb[t // 8, pl.ds((t % 8) * 16, 16)] = jnp.zeros((16,), jnp.float32)
        return carry

    lax.fori_loop(0, 16 * 8, fill_z, 0)

    def init(r, carry):
        pltpu.sync_copy(zb, acc.at[pl.ds(s * RPT + r * 16, 16)])
        return carry

    lax.fori_loop(0, RPT // 16, init, 0)

    @pl.when(s == NS - 1)
    def _():
        pltpu.sync_copy(zb, acc.at[pl.ds(NS * RPT, NTAIL)])

    plsc.subcore_barrier()

    def step(j, carry):
        chunk = wid + j * NWORK

        @pl.when(chunk < NCH)
        def _():
            pltpu.sync_copy(src_hbm.at[pl.ds(chunk * CH, CH)], src_v)
            pltpu.sync_copy(dst_hbm.at[pl.ds(chunk * CH, CH)], dst_v)
            pltpu.async_copy(y_hbm.at[src_v], rows_v, sem).wait()
            pltpu.sync_copy(rows_v, acc.at[dst_v], add=True)

        return carry

    lax.fori_loop(0, ITERS, step, 0)
    plsc.subcore_barrier()
    pltpu.sync_copy(acc.at[pl.ds(s * RPT, RPT)],
                    out_hbm.at[c, pl.ds(s * RPT, RPT)])

    @pl.when(s == NS - 1)
    def _():
        pltpu.sync_copy(acc.at[pl.ds(NS * RPT, NTAIL)],
                        out_hbm.at[c, pl.ds(NS * RPT, NTAIL)])


_scat_kernel = pl.kernel(
    _scat_body,
    out_type=jax.ShapeDtypeStruct((NC, N, D), jnp.float32),
    mesh=_mesh,
    scratch_types=[
        pltpu.VMEM((CH,), jnp.int32),
        pltpu.VMEM((CH,), jnp.int32),
        pltpu.VMEM((CH, D), jnp.float32),
        pltpu.VMEM((16, D), jnp.float32),
        pltpu.VMEM_SHARED((N, D), jnp.float32),
        pltpu.SemaphoreType.DMA,
    ],
)

_BLK = 1000
_GRID = N // _BLK


def _tc1_body(x_ref, w_ref, degp_ref, y_ref, dinv_ref):
    deg = jnp.sum(degp_ref[...], axis=0) + 1.0
    dinv = lax.rsqrt(deg)
    dinv_ref[...] = dinv
    xl = lax.dot_general(x_ref[...], w_ref[...], (((1,), (1,)), ((), ())),
                         preferred_element_type=jnp.float32)
    y_ref[...] = dinv * xl


_tc1 = pl.pallas_call(
    _tc1_body,
    grid=(_GRID,),
    in_specs=[
        pl.BlockSpec((_BLK, D), lambda i: (i, 0)),
        pl.BlockSpec((D, D), lambda i: (0, 0)),
        pl.BlockSpec((NWORK, _BLK, 1), lambda i: (0, i, 0)),
    ],
    out_specs=[
        pl.BlockSpec((_BLK, D), lambda i: (i, 0)),
        pl.BlockSpec((_BLK, 1), lambda i: (i, 0)),
    ],
    out_shape=[
        jax.ShapeDtypeStruct((N, D), jnp.float32),
        jax.ShapeDtypeStruct((N, 1), jnp.float32),
    ],
)


def _tc2_body(s_ref, y_ref, dinv_ref, b_ref, w_ref, y2_ref):
    dinv = dinv_ref[...]
    h = dinv * (s_ref[0] + s_ref[1] + y_ref[...]) + b_ref[...]
    h = jnp.where(h >= 0, h, 0.01 * h)
    y2_ref[...] = dinv * lax.dot_general(
        h, w_ref[...], (((1,), (1,)), ((), ())),
        preferred_element_type=jnp.float32)


_tc2 = pl.pallas_call(
    _tc2_body,
    grid=(_GRID,),
    in_specs=[
        pl.BlockSpec((NC, _BLK, D), lambda i: (0, i, 0)),
        pl.BlockSpec((_BLK, D), lambda i: (i, 0)),
        pl.BlockSpec((_BLK, 1), lambda i: (i, 0)),
        pl.BlockSpec((1, D), lambda i: (0, 0)),
        pl.BlockSpec((D, D), lambda i: (0, 0)),
    ],
    out_specs=pl.BlockSpec((_BLK, D), lambda i: (i, 0)),
    out_shape=jax.ShapeDtypeStruct((N, D), jnp.float32),
)


def _tc3_body(s_ref, y_ref, dinv_ref, b_ref, wd_ref, bd_ref, o_ref):
    dinv = dinv_ref[...]
    h = dinv * (s_ref[0] + s_ref[1] + y_ref[...]) + b_ref[...]
    h = jnp.where(h >= 0, h, 0.01 * h)
    o_ref[...] = jnp.sum(h * wd_ref[...], axis=1, keepdims=True) + bd_ref[0, 0]


_tc3 = pl.pallas_call(
    _tc3_body,
    grid=(_GRID,),
    in_specs=[
        pl.BlockSpec((NC, _BLK, D), lambda i: (0, i, 0)),
        pl.BlockSpec((_BLK, D), lambda i: (i, 0)),
        pl.BlockSpec((_BLK, 1), lambda i: (i, 0)),
        pl.BlockSpec((1, D), lambda i: (0, 0)),
        pl.BlockSpec((1, D), lambda i: (0, 0)),
        pl.BlockSpec(memory_space=pltpu.SMEM),
    ],
    out_specs=pl.BlockSpec((_BLK, 1), lambda i: (i, 0)),
    out_shape=jax.ShapeDtypeStruct((N, 1), jnp.float32),
)


def kernel(x, edge_index, W1, b1, W2, b2, Wd, bd):
    src = edge_index[0].astype(jnp.int32)
    dst = edge_index[1].astype(jnp.int32)
    degp = _deg_kernel(dst).reshape(NWORK, N, 1)
    y1, dinv = _tc1(x, W1, degp)
    s1 = _scat_kernel(src, dst, y1)
    y2 = _tc2(s1, y1, dinv, b1.reshape(1, D), W2)
    s2 = _scat_kernel(src, dst, y2)
    return _tc3(s2, y2, dinv, b2.reshape(1, D), Wd, bd.reshape(1, 1))


# speedup vs baseline: 15.7808x; 1.2451x over previous
import functools

import jax
import jax.numpy as jnp
from jax import lax
from jax.experimental import pallas as pl
from jax.experimental.pallas import tpu as pltpu
from jax.experimental.pallas import tpu_sc as plsc

N = 10000
E = 320000
D = 128
NC = 2
NS = 16
CH = 128
NCH = E // CH
NWORK = NC * NS
ITERS = (NCH + NWORK - 1) // NWORK
RPT = 624
NTAIL = N - NS * RPT

_mesh = plsc.VectorSubcoreMesh(core_axis_name="c", subcore_axis_name="s")


def _deg_body(dst_hbm, out_hbm, dst_v, hist):
    c = lax.axis_index("c")
    s = lax.axis_index("s")
    wid = c * NS + s

    def fill_z(i, carry):
        hist[pl.ds(i * 16, 16)] = jnp.zeros((16,), jnp.float32)
        return carry

    lax.fori_loop(0, N // 16, fill_z, 0)

    ones16 = jnp.ones((16,), jnp.float32)

    def step(j, carry):
        chunk = wid + j * NWORK

        @pl.when(chunk < NCH)
        def _():
            pltpu.sync_copy(dst_hbm.at[pl.ds(chunk * CH, CH)], dst_v)
            for k in range(CH // 16):
                idx = dst_v[pl.ds(k * 16, 16)]
                plsc.addupdate_scatter(hist, [idx], ones16)

        return carry

    lax.fori_loop(0, ITERS, step, 0)
    pltpu.sync_copy(hist, out_hbm.at[pl.ds(wid * N, N)])


_deg_kernel = pl.kernel(
    _deg_body,
    out_type=jax.ShapeDtypeStruct((NWORK * N,), jnp.float32),
    mesh=_mesh,
    scratch_types=[
        pltpu.VMEM((CH,), jnp.int32),
        pltpu.VMEM((N,), jnp.float32),
    ],
    compiler_params=pltpu.CompilerParams(needs_layout_passes=False),
)


NB = 2
NGRP = (ITERS + NB - 1) // NB


def _scat_body(pk_hbm, y_hbm, out_hbm, idx_v, rows_v, zb, acc, isem, gsem,
               ssem):
    c = lax.axis_index("c")
    s = lax.axis_index("s")
    wid = c * NS + s

    def fill_z(t, carry):
        zb[t // 8, pl.ds((t % 8) * 16, 16)] = jnp.zeros((16,), jnp.float32)
        return carry

    lax.fori_loop(0, 16 * 8, fill_z, 0)

    def init(r, carry):
        pltpu.sync_copy(zb, acc.at[pl.ds(s * RPT + r * 16, 16)])
        return carry

    lax.fori_loop(0, RPT // 16, init, 0)

    @pl.when(s == NS - 1)
    def _():
        pltpu.sync_copy(zb, acc.at[pl.ds(NS * RPT, NTAIL)])

    plsc.subcore_barrier()

    def group(g, carry):
        for b in range(NB):
            chunk = wid + (g * NB + b) * NWORK

            @pl.when(chunk < NCH)
            def _():
                pltpu.async_copy(pk_hbm.at[chunk], idx_v.at[b], isem.at[b])

        for b in range(NB):
            chunk = wid + (g * NB + b) * NWORK

            @pl.when(chunk < NCH)
            def _():
                pltpu.make_async_copy(pk_hbm.at[chunk], idx_v.at[b],
                                      isem.at[b]).wait()
                pltpu.async_copy(y_hbm.at[idx_v.at[b, 0]], rows_v.at[b],
                                 gsem.at[b])

        for b in range(NB):
            chunk = wid + (g * NB + b) * NWORK

            @pl.when(chunk < NCH)
            def _():
                pltpu.make_async_copy(y_hbm.at[idx_v.at[b, 0]], rows_v.at[b],
                                      gsem.at[b]).wait()
                pltpu.async_copy(rows_v.at[b], acc.at[idx_v.at[b, 1]],
                                 ssem.at[b], add=True)

        for b in range(NB):
            chunk = wid + (g * NB + b) * NWORK

            @pl.when(chunk < NCH)
            def _():
                pltpu.make_async_copy(rows_v.at[b], acc.at[idx_v.at[b, 1]],
                                      ssem.at[b]).wait()

        return carry

    lax.fori_loop(0, NGRP, group, 0)
    plsc.subcore_barrier()
    pltpu.sync_copy(acc.at[pl.ds(s * RPT, RPT)],
                    out_hbm.at[c, pl.ds(s * RPT, RPT)])

    @pl.when(s == NS - 1)
    def _():
        pltpu.sync_copy(acc.at[pl.ds(NS * RPT, NTAIL)],
                        out_hbm.at[c, pl.ds(NS * RPT, NTAIL)])


_scat_kernel = pl.kernel(
    _scat_body,
    out_type=jax.ShapeDtypeStruct((NC, N, D), jnp.float32),
    mesh=_mesh,
    scratch_types=[
        pltpu.VMEM((NB, 2, CH), jnp.int32),
        pltpu.VMEM((NB, CH, D), jnp.float32),
        pltpu.VMEM((16, D), jnp.float32),
        pltpu.VMEM_SHARED((N, D), jnp.float32),
        pltpu.SemaphoreType.DMA((NB,)),
        pltpu.SemaphoreType.DMA((NB,)),
        pltpu.SemaphoreType.DMA((NB,)),
    ],
)

_BLK = 1000
_GRID = N // _BLK


def _tc1_body(x_ref, w_ref, degp_ref, y_ref, dinv_ref):
    deg = jnp.sum(degp_ref[...], axis=0) + 1.0
    dinv = lax.rsqrt(deg)
    dinv_ref[...] = dinv
    xl = lax.dot_general(x_ref[...], w_ref[...], (((1,), (1,)), ((), ())),
                         preferred_element_type=jnp.float32)
    y_ref[...] = dinv * xl


_tc1 = pl.pallas_call(
    _tc1_body,
    grid=(_GRID,),
    in_specs=[
        pl.BlockSpec((_BLK, D), lambda i: (i, 0)),
        pl.BlockSpec((D, D), lambda i: (0, 0)),
        pl.BlockSpec((NWORK, _BLK, 1), lambda i: (0, i, 0)),
    ],
    out_specs=[
        pl.BlockSpec((_BLK, D), lambda i: (i, 0)),
        pl.BlockSpec((_BLK, 1), lambda i: (i, 0)),
    ],
    out_shape=[
        jax.ShapeDtypeStruct((N, D), jnp.float32),
        jax.ShapeDtypeStruct((N, 1), jnp.float32),
    ],
)


def _tc2_body(s_ref, y_ref, dinv_ref, b_ref, w_ref, y2_ref):
    dinv = dinv_ref[...]
    h = dinv * (s_ref[0] + s_ref[1] + y_ref[...]) + b_ref[...]
    h = jnp.where(h >= 0, h, 0.01 * h)
    y2_ref[...] = dinv * lax.dot_general(
        h, w_ref[...], (((1,), (1,)), ((), ())),
        preferred_element_type=jnp.float32)


_tc2 = pl.pallas_call(
    _tc2_body,
    grid=(_GRID,),
    in_specs=[
        pl.BlockSpec((NC, _BLK, D), lambda i: (0, i, 0)),
        pl.BlockSpec((_BLK, D), lambda i: (i, 0)),
        pl.BlockSpec((_BLK, 1), lambda i: (i, 0)),
        pl.BlockSpec((1, D), lambda i: (0, 0)),
        pl.BlockSpec((D, D), lambda i: (0, 0)),
    ],
    out_specs=pl.BlockSpec((_BLK, D), lambda i: (i, 0)),
    out_shape=jax.ShapeDtypeStruct((N, D), jnp.float32),
)


def _tc3_body(s_ref, y_ref, dinv_ref, b_ref, wd_ref, bd_ref, o_ref):
    dinv = dinv_ref[...]
    h = dinv * (s_ref[0] + s_ref[1] + y_ref[...]) + b_ref[...]
    h = jnp.where(h >= 0, h, 0.01 * h)
    o_ref[...] = jnp.sum(h * wd_ref[...], axis=1, keepdims=True) + bd_ref[0, 0]


_tc3 = pl.pallas_call(
    _tc3_body,
    grid=(_GRID,),
    in_specs=[
        pl.BlockSpec((NC, _BLK, D), lambda i: (0, i, 0)),
        pl.BlockSpec((_BLK, D), lambda i: (i, 0)),
        pl.BlockSpec((_BLK, 1), lambda i: (i, 0)),
        pl.BlockSpec((1, D), lambda i: (0, 0)),
        pl.BlockSpec((1, D), lambda i: (0, 0)),
        pl.BlockSpec(memory_space=pltpu.SMEM),
    ],
    out_specs=pl.BlockSpec((_BLK, 1), lambda i: (i, 0)),
    out_shape=jax.ShapeDtypeStruct((N, 1), jnp.float32),
)


def kernel(x, edge_index, W1, b1, W2, b2, Wd, bd):
    src = edge_index[0].astype(jnp.int32)
    dst = edge_index[1].astype(jnp.int32)
    packed = jnp.stack([src.reshape(NCH, CH), dst.reshape(NCH, CH)], axis=1)
    degp = _deg_kernel(dst).reshape(NWORK, N, 1)
    y1, dinv = _tc1(x, W1, degp)
    s1 = _scat_kernel(packed, y1)
    y2 = _tc2(s1, y1, dinv, b1.reshape(1, D), W2)
    s2 = _scat_kernel(packed, y2)
    return _tc3(s2, y2, dinv, b2.reshape(1, D), Wd, bd.reshape(1, 1))


# speedup vs baseline: 18.8910x; 1.1971x over previous
import functools

import jax
import jax.numpy as jnp
from jax import lax
from jax.experimental import pallas as pl
from jax.experimental.pallas import tpu as pltpu
from jax.experimental.pallas import tpu_sc as plsc

N = 10000
E = 320000
D = 128
NC = 2
NS = 16
CH = 128
NCH = E // CH
NWORK = NC * NS
ITERS = (NCH + NWORK - 1) // NWORK
RPT = 624
NTAIL = N - NS * RPT

_mesh = plsc.VectorSubcoreMesh(core_axis_name="c", subcore_axis_name="s")


CHD = 2000
EPW = E // NWORK


def _deg_body(dst_hbm, out_hbm, dst_v0, dst_v1, hist, isem):
    c = lax.axis_index("c")
    s = lax.axis_index("s")
    wid = c * NS + s
    bufs = (dst_v0, dst_v1)

    def fill_z(i, carry):
        hist[pl.ds(i * 16, 16)] = jnp.zeros((16,), jnp.float32)
        return carry

    lax.fori_loop(0, N // 16, fill_z, 0)

    ones16 = jnp.ones((16,), jnp.float32)
    nstep = EPW // CHD

    pltpu.async_copy(dst_hbm.at[pl.ds(wid * EPW, CHD)], bufs[0], isem.at[0])
    for j in range(nstep):
        b = j % 2
        if j + 1 < nstep:
            pltpu.async_copy(dst_hbm.at[pl.ds(wid * EPW + (j + 1) * CHD, CHD)],
                             bufs[1 - b], isem.at[1 - b])
        pltpu.make_async_copy(dst_hbm.at[pl.ds(wid * EPW + j * CHD, CHD)],
                              bufs[b], isem.at[b]).wait()

        def upd(k, carry2, _buf=bufs[b]):
            idx = _buf[pl.ds(k * 16, 16)]
            plsc.addupdate_scatter(hist, [idx], ones16)
            return carry2

        lax.fori_loop(0, CHD // 16, upd, 0)

    pltpu.sync_copy(hist, out_hbm.at[pl.ds(wid * N, N)])


_deg_kernel = pl.kernel(
    _deg_body,
    out_type=jax.ShapeDtypeStruct((NWORK * N,), jnp.float32),
    mesh=_mesh,
    scratch_types=[
        pltpu.VMEM((CHD,), jnp.int32),
        pltpu.VMEM((CHD,), jnp.int32),
        pltpu.VMEM((N,), jnp.float32),
        pltpu.SemaphoreType.DMA((2,)),
    ],
    compiler_params=pltpu.CompilerParams(needs_layout_passes=False),
)


NB = 2
NGRP = (ITERS + NB - 1) // NB


def _scat_body(pk_hbm, y_hbm, out_hbm, idx_v, rows_v, zb, acc, isem, gsem,
               ssem):
    c = lax.axis_index("c")
    s = lax.axis_index("s")
    wid = c * NS + s

    def fill_z(t, carry):
        zb[t // 8, pl.ds((t % 8) * 16, 16)] = jnp.zeros((16,), jnp.float32)
        return carry

    lax.fori_loop(0, 16 * 8, fill_z, 0)

    def init(r, carry):
        pltpu.async_copy(zb, acc.at[pl.ds(s * RPT + r * 16, 16)], isem.at[0])
        return carry

    lax.fori_loop(0, RPT // 16, init, 0)

    @pl.when(s == NS - 1)
    def _():
        pltpu.async_copy(zb, acc.at[pl.ds(NS * RPT, NTAIL)], isem.at[0])

    def init_drain(r, carry):
        pltpu.make_async_copy(zb, acc.at[pl.ds(s * RPT + r * 16, 16)],
                              isem.at[0]).wait()
        return carry

    lax.fori_loop(0, RPT // 16, init_drain, 0)

    @pl.when(s == NS - 1)
    def _():
        pltpu.make_async_copy(zb, acc.at[pl.ds(NS * RPT, NTAIL)],
                              isem.at[0]).wait()

    plsc.subcore_barrier()

    def drain_scat(b):
        pltpu.make_async_copy(rows_v.at[b], acc.at[idx_v.at[0, 1]],
                              ssem.at[b]).wait()

    def emit_half(t, half):
        base = (t * 2 + half) * 4

        def I(u):
            chunk = wid + (base + u) * NWORK

            @pl.when(chunk < NCH)
            def _():
                pltpu.async_copy(pk_hbm.at[chunk], idx_v.at[half * 4 + u],
                                 isem.at[half * 4 + u])

        def G(u):
            chunk = wid + (base + u) * NWORK
            q = half * 4 + u
            b = u % 2

            @pl.when(chunk < NCH)
            def _():
                pltpu.make_async_copy(pk_hbm.at[chunk], idx_v.at[q],
                                      isem.at[q]).wait()
                if u < 2 and half == 0:
                    @pl.when(t > 0)
                    def _():
                        drain_scat(b)
                else:
                    drain_scat(b)
                pltpu.async_copy(y_hbm.at[idx_v.at[q, 0]], rows_v.at[b],
                                 gsem.at[b])

        def S(u):
            chunk = wid + (base + u) * NWORK
            q = half * 4 + u
            b = u % 2

            @pl.when(chunk < NCH)
            def _():
                pltpu.make_async_copy(y_hbm.at[idx_v.at[q, 0]], rows_v.at[b],
                                      gsem.at[b]).wait()
                pltpu.async_copy(rows_v.at[b], acc.at[idx_v.at[q, 1]],
                                 ssem.at[b], add=True)

        I(0); I(1); I(2); I(3)
        G(0); G(1); S(0); G(2); S(1); G(3); S(2); S(3)

    def group(t, carry):
        emit_half(t, 0)
        emit_half(t, 1)
        return carry

    lax.fori_loop(0, (ITERS + 7) // 8, group, 0)
    for b in range(NB):
        drain_scat(b)
    plsc.subcore_barrier()
    pltpu.sync_copy(acc.at[pl.ds(s * RPT, RPT)],
                    out_hbm.at[c, pl.ds(s * RPT, RPT)])

    @pl.when(s == NS - 1)
    def _():
        pltpu.sync_copy(acc.at[pl.ds(NS * RPT, NTAIL)],
                        out_hbm.at[c, pl.ds(NS * RPT, NTAIL)])


_scat_kernel = pl.kernel(
    _scat_body,
    out_type=jax.ShapeDtypeStruct((NC, N, D), jnp.float32),
    mesh=_mesh,
    scratch_types=[
        pltpu.VMEM((8, 2, CH), jnp.int32),
        pltpu.VMEM((NB, CH, D), jnp.float32),
        pltpu.VMEM((16, D), jnp.float32),
        pltpu.VMEM_SHARED((N, D), jnp.float32),
        pltpu.SemaphoreType.DMA((8,)),
        pltpu.SemaphoreType.DMA((NB,)),
        pltpu.SemaphoreType.DMA((NB,)),
    ],
)

_BLK = 1000
_GRID = N // _BLK


def _tcmm_body(x_ref, w_ref, o_ref):
    o_ref[...] = lax.dot_general(x_ref[...], w_ref[...],
                                 (((1,), (1,)), ((), ())),
                                 preferred_element_type=jnp.float32)


_tc_mm = pl.pallas_call(
    _tcmm_body,
    grid=(_GRID,),
    in_specs=[
        pl.BlockSpec((_BLK, D), lambda i: (i, 0)),
        pl.BlockSpec((D, D), lambda i: (0, 0)),
    ],
    out_specs=pl.BlockSpec((_BLK, D), lambda i: (i, 0)),
    out_shape=jax.ShapeDtypeStruct((N, D), jnp.float32),
)


def _tcscale_body(xl_ref, degp_ref, y_ref, dinv_ref):
    deg = jnp.sum(degp_ref[...], axis=0) + 1.0
    d0 = lax.rsqrt(deg)
    dinv = d0 * (1.5 - 0.5 * deg * d0 * d0)
    dinv_ref[...] = dinv
    y_ref[...] = dinv * xl_ref[...]


_tc_scale = pl.pallas_call(
    _tcscale_body,
    grid=(_GRID,),
    in_specs=[
        pl.BlockSpec((_BLK, D), lambda i: (i, 0)),
        pl.BlockSpec((NWORK, _BLK, 1), lambda i: (0, i, 0)),
    ],
    out_specs=[
        pl.BlockSpec((_BLK, D), lambda i: (i, 0)),
        pl.BlockSpec((_BLK, 1), lambda i: (i, 0)),
    ],
    out_shape=[
        jax.ShapeDtypeStruct((N, D), jnp.float32),
        jax.ShapeDtypeStruct((N, 1), jnp.float32),
    ],
)


def _tc2_body(s_ref, y_ref, dinv_ref, b_ref, w_ref, y2_ref):
    dinv = dinv_ref[...]
    h = dinv * (s_ref[0] + s_ref[1] + y_ref[...]) + b_ref[...]
    h = jnp.where(h >= 0, h, 0.01 * h)
    y2_ref[...] = dinv * lax.dot_general(
        h, w_ref[...], (((1,), (1,)), ((), ())),
        preferred_element_type=jnp.float32)


_tc2 = pl.pallas_call(
    _tc2_body,
    grid=(_GRID,),
    in_specs=[
        pl.BlockSpec((NC, _BLK, D), lambda i: (0, i, 0)),
        pl.BlockSpec((_BLK, D), lambda i: (i, 0)),
        pl.BlockSpec((_BLK, 1), lambda i: (i, 0)),
        pl.BlockSpec((1, D), lambda i: (0, 0)),
        pl.BlockSpec((D, D), lambda i: (0, 0)),
    ],
    out_specs=pl.BlockSpec((_BLK, D), lambda i: (i, 0)),
    out_shape=jax.ShapeDtypeStruct((N, D), jnp.float32),
)


def _tc3_body(s_ref, y_ref, dinv_ref, b_ref, wd_ref, bd_ref, o_ref):
    dinv = dinv_ref[...]
    h = dinv * (s_ref[0] + s_ref[1] + y_ref[...]) + b_ref[...]
    h = jnp.where(h >= 0, h, 0.01 * h)
    r = lax.dot_general(h, wd_ref[...], (((1,), (1,)), ((), ())),
                        preferred_element_type=jnp.float32)
    o_ref[...] = r[:, 0:1] + bd_ref[0, 0]


_tc3 = pl.pallas_call(
    _tc3_body,
    grid=(_GRID,),
    in_specs=[
        pl.BlockSpec((NC, _BLK, D), lambda i: (0, i, 0)),
        pl.BlockSpec((_BLK, D), lambda i: (i, 0)),
        pl.BlockSpec((_BLK, 1), lambda i: (i, 0)),
        pl.BlockSpec((1, D), lambda i: (0, 0)),
        pl.BlockSpec((D, D), lambda i: (0, 0)),
        pl.BlockSpec(memory_space=pltpu.SMEM),
    ],
    out_specs=pl.BlockSpec((_BLK, 1), lambda i: (i, 0)),
    out_shape=jax.ShapeDtypeStruct((N, 1), jnp.float32),
)


def kernel(x, edge_index, W1, b1, W2, b2, Wd, bd):
    src = edge_index[0].astype(jnp.int32)
    dst = edge_index[1].astype(jnp.int32)
    packed = jnp.stack([src.reshape(NCH, CH), dst.reshape(NCH, CH)], axis=1)
    degp = _deg_kernel(dst).reshape(NWORK, N, 1)
    xl1 = _tc_mm(x, W1)
    y1, dinv = _tc_scale(xl1, degp)
    s1 = _scat_kernel(packed, y1)
    y2 = _tc2(s1, y1, dinv, b1.reshape(1, D), W2)
    s2 = _scat_kernel(packed, y2)
    wd_pad = jnp.zeros((D, D), jnp.float32).at[0].set(Wd[0])
    return _tc3(s2, y2, dinv, b2.reshape(1, D), wd_pad, bd.reshape(1, 1))


# speedup vs baseline: 19.5375x; 1.0342x over previous
import functools

import jax
import jax.numpy as jnp
from jax import lax
from jax.experimental import pallas as pl
from jax.experimental.pallas import tpu as pltpu
from jax.experimental.pallas import tpu_sc as plsc

N = 10000
E = 320000
D = 128
NC = 2
NS = 16
CH = 128
NCH = E // CH
NWORK = NC * NS
ITERS = (NCH + NWORK - 1) // NWORK
RPT = 624
NTAIL = N - NS * RPT

_mesh = plsc.VectorSubcoreMesh(core_axis_name="c", subcore_axis_name="s")


CHD = 2000
EPW = E // NWORK


def _deg_body(ei_hbm, out_hbm, dst_v0, dst_v1, hist, isem):
    c = lax.axis_index("c")
    s = lax.axis_index("s")
    wid = c * NS + s
    bufs = (dst_v0, dst_v1)

    def fill_z(i, carry):
        hist[pl.ds(i * 16, 16)] = jnp.zeros((16,), jnp.float32)
        return carry

    lax.fori_loop(0, N // 16, fill_z, 0)

    ones16 = jnp.ones((16,), jnp.float32)
    nstep = EPW // CHD

    base0 = E + wid * EPW
    pltpu.async_copy(ei_hbm.at[pl.ds(base0, CHD)], bufs[0], isem.at[0])
    for j in range(nstep):
        b = j % 2
        if j + 1 < nstep:
            pltpu.async_copy(ei_hbm.at[pl.ds(base0 + (j + 1) * CHD, CHD)],
                             bufs[1 - b], isem.at[1 - b])
        pltpu.make_async_copy(ei_hbm.at[pl.ds(base0 + j * CHD, CHD)],
                              bufs[b], isem.at[b]).wait()

        def upd(k, carry2, _buf=bufs[b]):
            idx = _buf[pl.ds(k * 16, 16)]
            plsc.addupdate_scatter(hist, [idx], ones16)
            return carry2

        lax.fori_loop(0, CHD // 16, upd, 0)

    pltpu.sync_copy(hist, out_hbm.at[pl.ds(wid * N, N)])


_deg_kernel = pl.kernel(
    _deg_body,
    out_type=jax.ShapeDtypeStruct((NWORK * N,), jnp.float32),
    mesh=_mesh,
    scratch_types=[
        pltpu.VMEM((CHD,), jnp.int32),
        pltpu.VMEM((CHD,), jnp.int32),
        pltpu.VMEM((N,), jnp.float32),
        pltpu.SemaphoreType.DMA((2,)),
    ],
    compiler_params=pltpu.CompilerParams(needs_layout_passes=False),
)


NB = 2
NGRP = (ITERS + NB - 1) // NB


def _scat_body(ei_hbm, y_hbm, out_hbm, idx_v, rows_v, zb, acc, isem, gsem,
               ssem):
    c = lax.axis_index("c")
    s = lax.axis_index("s")
    wid = c * NS + s

    def fill_z(t, carry):
        zb[t // 8, pl.ds((t % 8) * 16, 16)] = jnp.zeros((16,), jnp.float32)
        return carry

    lax.fori_loop(0, 16 * 8, fill_z, 0)

    def init(r, carry):
        pltpu.async_copy(zb, acc.at[pl.ds(s * RPT + r * 16, 16)], isem.at[0])
        return carry

    lax.fori_loop(0, RPT // 16, init, 0)

    @pl.when(s == NS - 1)
    def _():
        pltpu.async_copy(zb, acc.at[pl.ds(NS * RPT, NTAIL)], isem.at[0])

    def init_drain(r, carry):
        pltpu.make_async_copy(zb, acc.at[pl.ds(s * RPT + r * 16, 16)],
                              isem.at[0]).wait()
        return carry

    lax.fori_loop(0, RPT // 16, init_drain, 0)

    @pl.when(s == NS - 1)
    def _():
        pltpu.make_async_copy(zb, acc.at[pl.ds(NS * RPT, NTAIL)],
                              isem.at[0]).wait()

    plsc.subcore_barrier()

    def drain_scat(b):
        pltpu.make_async_copy(rows_v.at[b], acc.at[idx_v.at[0, 1]],
                              ssem.at[b]).wait()

    def emit_half(t, half):
        base = (t * 2 + half) * 4

        def I(u):
            chunk = wid + (base + u) * NWORK
            q = half * 4 + u

            @pl.when(chunk < NCH)
            def _():
                pltpu.async_copy(ei_hbm.at[pl.ds(chunk * CH, CH)],
                                 idx_v.at[q, 0], isem.at[q])
                pltpu.async_copy(ei_hbm.at[pl.ds(E + chunk * CH, CH)],
                                 idx_v.at[q, 1], isem.at[q])

        def G(u):
            chunk = wid + (base + u) * NWORK
            q = half * 4 + u
            b = u % 2

            @pl.when(chunk < NCH)
            def _():
                pltpu.make_async_copy(ei_hbm.at[pl.ds(chunk * CH, CH)],
                                      idx_v.at[q, 0], isem.at[q]).wait()
                pltpu.make_async_copy(ei_hbm.at[pl.ds(E + chunk * CH, CH)],
                                      idx_v.at[q, 1], isem.at[q]).wait()
                if u < 2 and half == 0:
                    @pl.when(t > 0)
                    def _():
                        drain_scat(b)
                else:
                    drain_scat(b)
                pltpu.async_copy(y_hbm.at[idx_v.at[q, 0]], rows_v.at[b],
                                 gsem.at[b])

        def S(u):
            chunk = wid + (base + u) * NWORK
            q = half * 4 + u
            b = u % 2

            @pl.when(chunk < NCH)
            def _():
                pltpu.make_async_copy(y_hbm.at[idx_v.at[q, 0]], rows_v.at[b],
                                      gsem.at[b]).wait()
                pltpu.async_copy(rows_v.at[b], acc.at[idx_v.at[q, 1]],
                                 ssem.at[b], add=True)

        I(0); I(1); I(2); I(3)
        G(0); G(1); S(0); G(2); S(1); G(3); S(2); S(3)

    def group(t, carry):
        emit_half(t, 0)
        emit_half(t, 1)
        return carry

    lax.fori_loop(0, (ITERS + 7) // 8, group, 0)
    for b in range(NB):
        drain_scat(b)
    plsc.subcore_barrier()
    pltpu.sync_copy(acc.at[pl.ds(s * RPT, RPT)],
                    out_hbm.at[c, pl.ds(s * RPT, RPT)])

    @pl.when(s == NS - 1)
    def _():
        pltpu.sync_copy(acc.at[pl.ds(NS * RPT, NTAIL)],
                        out_hbm.at[c, pl.ds(NS * RPT, NTAIL)])


_scat_kernel = pl.kernel(
    _scat_body,
    out_type=jax.ShapeDtypeStruct((NC, N, D), jnp.float32),
    mesh=_mesh,
    scratch_types=[
        pltpu.VMEM((8, 2, CH), jnp.int32),
        pltpu.VMEM((NB, CH, D), jnp.float32),
        pltpu.VMEM((16, D), jnp.float32),
        pltpu.VMEM_SHARED((N, D), jnp.float32),
        pltpu.SemaphoreType.DMA((8,)),
        pltpu.SemaphoreType.DMA((NB,)),
        pltpu.SemaphoreType.DMA((NB,)),
    ],
)

_BLK = 1000
_GRID = N // _BLK


def _tc1_body(x_ref, w_ref, degp_ref, y_ref, dinv_ref):
    deg = jnp.sum(degp_ref[...], axis=0) + 1.0
    d0 = lax.rsqrt(deg)
    dinv = d0 * (1.5 - 0.5 * deg * d0 * d0)
    dinv_ref[...] = dinv
    y_ref[...] = dinv * lax.dot_general(x_ref[...], w_ref[...],
                                        (((1,), (1,)), ((), ())),
                                        preferred_element_type=jnp.float32)


_tc1 = pl.pallas_call(
    _tc1_body,
    grid=(_GRID,),
    in_specs=[
        pl.BlockSpec((_BLK, D), lambda i: (i, 0)),
        pl.BlockSpec((D, D), lambda i: (0, 0)),
        pl.BlockSpec((NWORK, _BLK, 1), lambda i: (0, i, 0)),
    ],
    out_specs=[
        pl.BlockSpec((_BLK, D), lambda i: (i, 0)),
        pl.BlockSpec((_BLK, 1), lambda i: (i, 0)),
    ],
    out_shape=[
        jax.ShapeDtypeStruct((N, D), jnp.float32),
        jax.ShapeDtypeStruct((N, 1), jnp.float32),
    ],
)


def _tc2_body(s_ref, y_ref, dinv_ref, b_ref, w_ref, y2_ref):
    dinv = dinv_ref[...]
    h = dinv * (s_ref[0] + s_ref[1] + y_ref[...]) + b_ref[...]
    h = jnp.where(h >= 0, h, 0.01 * h)
    y2_ref[...] = dinv * lax.dot_general(
        h, w_ref[...], (((1,), (1,)), ((), ())),
        preferred_element_type=jnp.float32)


_tc2 = pl.pallas_call(
    _tc2_body,
    grid=(_GRID,),
    in_specs=[
        pl.BlockSpec((NC, _BLK, D), lambda i: (0, i, 0)),
        pl.BlockSpec((_BLK, D), lambda i: (i, 0)),
        pl.BlockSpec((_BLK, 1), lambda i: (i, 0)),
        pl.BlockSpec((1, D), lambda i: (0, 0)),
        pl.BlockSpec((D, D), lambda i: (0, 0)),
    ],
    out_specs=pl.BlockSpec((_BLK, D), lambda i: (i, 0)),
    out_shape=jax.ShapeDtypeStruct((N, D), jnp.float32),
)


def _tc3_body(s_ref, y_ref, dinv_ref, b_ref, wd_ref, bd_ref, o_ref):
    dinv = dinv_ref[...]
    h = dinv * (s_ref[0] + s_ref[1] + y_ref[...]) + b_ref[...]
    h = jnp.where(h >= 0, h, 0.01 * h)
    r = lax.dot_general(h, wd_ref[...], (((1,), (1,)), ((), ())),
                        preferred_element_type=jnp.float32)
    o_ref[...] = r[:, 0:1] + bd_ref[0, 0]


_tc3 = pl.pallas_call(
    _tc3_body,
    grid=(_GRID,),
    in_specs=[
        pl.BlockSpec((NC, _BLK, D), lambda i: (0, i, 0)),
        pl.BlockSpec((_BLK, D), lambda i: (i, 0)),
        pl.BlockSpec((_BLK, 1), lambda i: (i, 0)),
        pl.BlockSpec((1, D), lambda i: (0, 0)),
        pl.BlockSpec((D, D), lambda i: (0, 0)),
        pl.BlockSpec(memory_space=pltpu.SMEM),
    ],
    out_specs=pl.BlockSpec((_BLK, 1), lambda i: (i, 0)),
    out_shape=jax.ShapeDtypeStruct((N, 1), jnp.float32),
)


def kernel(x, edge_index, W1, b1, W2, b2, Wd, bd):
    ei = edge_index.astype(jnp.int32).reshape(2 * E)
    degp = _deg_kernel(ei).reshape(NWORK, N, 1)
    y1, dinv = _tc1(x, W1, degp)
    s1 = _scat_kernel(ei, y1)
    y2 = _tc2(s1, y1, dinv, b1.reshape(1, D), W2)
    s2 = _scat_kernel(ei, y2)
    wd_pad = jnp.zeros((D, D), jnp.float32).at[0].set(Wd[0])
    return _tc3(s2, y2, dinv, b2.reshape(1, D), wd_pad, bd.reshape(1, 1))
